# initial kernel scaffold (unmeasured)
import jax
import jax.numpy as jnp
from jax import lax
from jax.experimental import pallas as pl
from jax.experimental.pallas import tpu as pltpu

N_DEV = 4
S = 2048
H = 8
D = 128
HD = H * D
T = 1024
NT = S // T
BLK = 64
BPS = S // BLK
SCALE = 0.08838834764831843
NEG = -1e9


def _body(q_ref, k_ref, v_ref, bias_ref, ctx_ref,
          comm_k, comm_v, bias_vmem,
          ksend, krecv, vsend, vrecv, local_sems):
    my = lax.axis_index("i")
    left = lax.rem(my + N_DEV - 1, N_DEV)
    right = lax.rem(my + 1, N_DEV)

    barrier = pltpu.get_barrier_semaphore()
    for nbr in (left, right):
        pl.semaphore_signal(barrier, inc=1, device_id=(nbr,),
                            device_id_type=pl.DeviceIdType.MESH)
    pl.semaphore_wait(barrier, 2)

    ck = pltpu.make_async_copy(k_ref, comm_k.at[0], local_sems.at[0])
    cv = pltpu.make_async_copy(v_ref, comm_v.at[0], local_sems.at[1])
    cb = pltpu.make_async_copy(bias_ref.at[0], bias_vmem, local_sems.at[2])
    ck.start()
    cv.start()
    cb.start()
    ck.wait()
    cv.wait()
    cb.wait()

    m = [[None] * NT for _ in range(H)]
    l = [[None] * NT for _ in range(H)]
    acc = [[None] * NT for _ in range(H)]

    for s in range(N_DEV):
        rk = rv = None
        if s < N_DEV - 1:
            rk = pltpu.make_async_remote_copy(
                src_ref=comm_k.at[s % 2], dst_ref=comm_k.at[(s + 1) % 2],
                send_sem=ksend.at[s], recv_sem=krecv.at[s],
                device_id=(right,), device_id_type=pl.DeviceIdType.MESH)
            rv = pltpu.make_async_remote_copy(
                src_ref=comm_v.at[s % 2], dst_ref=comm_v.at[(s + 1) % 2],
                send_sem=vsend.at[s], recv_sem=vrecv.at[s],
                device_id=(right,), device_id_type=pl.DeviceIdType.MESH)
            rk.start()
            rv.start()

        slot = 0 if s == 0 else s % 2
        for h in range(H):
            k_h = comm_k[slot, h]
            v_h = comm_v[slot, h]
            for t in range(NT):
                q_t = q_ref[h, pl.ds(t * T, T), :]
                sc = lax.dot_general(
                    q_t, k_h, (((1,), (1,)), ((), ())),
                    preferred_element_type=jnp.float32)
                sc = sc * SCALE + bias_vmem[pl.ds(t * T, T), :].astype(
                    jnp.float32)
                mx = jnp.max(sc, axis=1, keepdims=True)
                if s == 0:
                    m_new = mx
                    w = jnp.exp(sc - m_new)
                    l_new = jnp.sum(w, axis=1, keepdims=True)
                    a_new = lax.dot_general(
                        w.astype(jnp.bfloat16), v_h,
                        (((1,), (0,)), ((), ())),
                        preferred_element_type=jnp.float32)
                else:
                    m_new = jnp.maximum(m[h][t], mx)
                    alpha = jnp.exp(m[h][t] - m_new)
                    w = jnp.exp(sc - m_new)
                    l_new = l[h][t] * alpha + jnp.sum(w, axis=1, keepdims=True)
                    a_new = acc[h][t] * alpha + lax.dot_general(
                        w.astype(jnp.bfloat16), v_h,
                        (((1,), (0,)), ((), ())),
                        preferred_element_type=jnp.float32)
                m[h][t], l[h][t], acc[h][t] = m_new, l_new, a_new

        if s < N_DEV - 1:
            rk.wait()
            rv.wait()
            cb = pltpu.make_async_copy(bias_ref.at[s + 1], bias_vmem,
                                       local_sems.at[2])
            cb.start()
            cb.wait()

    for h in range(H):
        for t in range(NT):
            inv = 1.0 / l[h][t]
            ctx_ref[pl.ds(t * T, T), h * D:(h + 1) * D] = (
                acc[h][t] * inv).astype(jnp.bfloat16)


def kernel(x, Wq, K_ext, V_ext, Wo):
    xb = x[0].astype(jnp.bfloat16)
    wqb = Wq.astype(jnp.bfloat16)
    q = lax.dot_general(xb, wqb, (((1,), (0,)), ((), ())),
                        preferred_element_type=jnp.float32)
    q = q.astype(jnp.bfloat16).reshape(S, H, D).transpose(1, 0, 2)
    kh = K_ext[0].astype(jnp.bfloat16).transpose(1, 0, 2)
    vh = V_ext[0].astype(jnp.bfloat16).transpose(1, 0, 2)

    my = lax.axis_index("i")
    qb = my * BPS + jnp.arange(S, dtype=jnp.int32) // BLK
    p_s = jnp.mod(my - jnp.arange(N_DEV, dtype=jnp.int32), N_DEV)
    kb = (p_s[:, None] * BPS
          + (jnp.arange(S, dtype=jnp.int32) // BLK)[None, :])
    qb2 = qb[None, :, None]
    kb2 = kb[:, None, :]
    mask = (qb2 == kb2) | (kb2 == 0) | ((qb2 + kb2) % 3 == 0)
    bias = jnp.where(mask, 0.0, NEG).astype(jnp.bfloat16)

    ctx = pl.pallas_call(
        _body,
        out_shape=jax.ShapeDtypeStruct((S, HD), jnp.bfloat16),
        in_specs=[
            pl.BlockSpec(memory_space=pltpu.VMEM),
            pl.BlockSpec(memory_space=pltpu.ANY),
            pl.BlockSpec(memory_space=pltpu.ANY),
            pl.BlockSpec(memory_space=pltpu.ANY),
        ],
        out_specs=pl.BlockSpec(memory_space=pltpu.VMEM),
        scratch_shapes=[
            pltpu.VMEM((2, H, S, D), jnp.bfloat16),
            pltpu.VMEM((2, H, S, D), jnp.bfloat16),
            pltpu.VMEM((S, S), jnp.bfloat16),
            pltpu.SemaphoreType.DMA((N_DEV - 1,)),
            pltpu.SemaphoreType.DMA((N_DEV - 1,)),
            pltpu.SemaphoreType.DMA((N_DEV - 1,)),
            pltpu.SemaphoreType.DMA((N_DEV - 1,)),
            pltpu.SemaphoreType.DMA((3,)),
        ],
        compiler_params=pltpu.CompilerParams(collective_id=0),
    )(q, kh, vh, bias)

    wob = Wo.astype(jnp.bfloat16)
    out = lax.dot_general(ctx, wob, (((1,), (0,)), ((), ())),
                          preferred_element_type=jnp.float32)
    return out.reshape(1, S, HD)


# baseline (device time: 456266 ns/iter reference)
import jax
import jax.numpy as jnp
from jax import lax
from jax.experimental import pallas as pl
from jax.experimental.pallas import tpu as pltpu

N_DEV = 4
S = 2048
H = 8
D = 128
HD = H * D
T = 512
NT = S // T
BLK = 64
BPS = S // BLK
SCALE = 0.08838834764831843
NEG = -1e9


def _body(q_ref, k_ref, v_ref, bias_ref, ctx_ref,
          comm_k, comm_v, bias_vmem, m_scr, l_scr, acc_scr,
          ksend, krecv, vsend, vrecv, local_sems):
    my = lax.axis_index("i")
    left = lax.rem(my + N_DEV - 1, N_DEV)
    right = lax.rem(my + 1, N_DEV)

    barrier = pltpu.get_barrier_semaphore()
    for nbr in (left, right):
        pl.semaphore_signal(barrier, inc=1, device_id=(nbr,),
                            device_id_type=pl.DeviceIdType.MESH)
    pl.semaphore_wait(barrier, 2)

    ck = pltpu.make_async_copy(k_ref, comm_k.at[0], local_sems.at[0])
    cv = pltpu.make_async_copy(v_ref, comm_v.at[0], local_sems.at[1])
    cb = pltpu.make_async_copy(bias_ref.at[0], bias_vmem, local_sems.at[2])
    ck.start()
    cv.start()
    cb.start()
    ck.wait()
    cv.wait()
    cb.wait()

    m_scr[...] = jnp.full((S, H), -1e30, jnp.float32)
    l_scr[...] = jnp.zeros((S, H), jnp.float32)
    acc_scr[...] = jnp.zeros((S, HD), jnp.float32)

    for s in range(N_DEV):
        rk = rv = None
        if s < N_DEV - 1:
            rk = pltpu.make_async_remote_copy(
                src_ref=comm_k.at[s % 2], dst_ref=comm_k.at[(s + 1) % 2],
                send_sem=ksend.at[s], recv_sem=krecv.at[s],
                device_id=(right,), device_id_type=pl.DeviceIdType.MESH)
            rv = pltpu.make_async_remote_copy(
                src_ref=comm_v.at[s % 2], dst_ref=comm_v.at[(s + 1) % 2],
                send_sem=vsend.at[s], recv_sem=vrecv.at[s],
                device_id=(right,), device_id_type=pl.DeviceIdType.MESH)
            rk.start()
            rv.start()

        slot = 0 if s == 0 else s % 2

        for h in range(H):
            def blk(t, carry, slot=slot, h=h):
                rows = pl.ds(t * T, T)
                q_t = q_ref[h, rows, :]
                k_h = comm_k[slot, h]
                v_h = comm_v[slot, h]
                sc = lax.dot_general(
                    q_t, k_h, (((1,), (1,)), ((), ())),
                    preferred_element_type=jnp.float32)
                sc = sc * SCALE + bias_vmem[rows, :].astype(jnp.float32)
                mx = jnp.max(sc, axis=1, keepdims=True)
                m_old = m_scr[rows, h:h + 1]
                m_new = jnp.maximum(m_old, mx)
                alpha = jnp.exp(m_old - m_new)
                w = jnp.exp(sc - m_new)
                l_new = (l_scr[rows, h:h + 1] * alpha
                         + jnp.sum(w, axis=1, keepdims=True))
                a_new = acc_scr[rows, h * D:(h + 1) * D] * alpha + \
                    lax.dot_general(
                        w.astype(jnp.bfloat16), v_h,
                        (((1,), (0,)), ((), ())),
                        preferred_element_type=jnp.float32)
                m_scr[rows, h:h + 1] = m_new
                l_scr[rows, h:h + 1] = l_new
                acc_scr[rows, h * D:(h + 1) * D] = a_new
                return carry

            lax.fori_loop(0, NT, blk, 0)

        if s < N_DEV - 1:
            rk.wait()
            rv.wait()
            cb = pltpu.make_async_copy(bias_ref.at[s + 1], bias_vmem,
                                       local_sems.at[2])
            cb.start()
            cb.wait()

    for h in range(H):
        inv = 1.0 / l_scr[:, h:h + 1]
        ctx_ref[:, h * D:(h + 1) * D] = (
            acc_scr[:, h * D:(h + 1) * D] * inv).astype(jnp.bfloat16)


def kernel(x, Wq, K_ext, V_ext, Wo):
    xb = x[0].astype(jnp.bfloat16)
    wqb = Wq.astype(jnp.bfloat16)
    q = lax.dot_general(xb, wqb, (((1,), (0,)), ((), ())),
                        preferred_element_type=jnp.float32)
    q = q.astype(jnp.bfloat16).reshape(S, H, D).transpose(1, 0, 2)
    kh = K_ext[0].astype(jnp.bfloat16).transpose(1, 0, 2)
    vh = V_ext[0].astype(jnp.bfloat16).transpose(1, 0, 2)

    my = lax.axis_index("i")
    qb = my * BPS + jnp.arange(S, dtype=jnp.int32) // BLK
    p_s = jnp.mod(my - jnp.arange(N_DEV, dtype=jnp.int32), N_DEV)
    kb = (p_s[:, None] * BPS
          + (jnp.arange(S, dtype=jnp.int32) // BLK)[None, :])
    qb2 = qb[None, :, None]
    kb2 = kb[:, None, :]
    mask = (qb2 == kb2) | (kb2 == 0) | ((qb2 + kb2) % 3 == 0)
    bias = jnp.where(mask, 0.0, NEG).astype(jnp.bfloat16)

    ctx = pl.pallas_call(
        _body,
        out_shape=jax.ShapeDtypeStruct((S, HD), jnp.bfloat16),
        in_specs=[
            pl.BlockSpec(memory_space=pltpu.VMEM),
            pl.BlockSpec(memory_space=pl.ANY),
            pl.BlockSpec(memory_space=pl.ANY),
            pl.BlockSpec(memory_space=pl.ANY),
        ],
        out_specs=pl.BlockSpec(memory_space=pltpu.VMEM),
        scratch_shapes=[
            pltpu.VMEM((2, H, S, D), jnp.bfloat16),
            pltpu.VMEM((2, H, S, D), jnp.bfloat16),
            pltpu.VMEM((S, S), jnp.bfloat16),
            pltpu.VMEM((S, H), jnp.float32),
            pltpu.VMEM((S, H), jnp.float32),
            pltpu.VMEM((S, HD), jnp.float32),
            pltpu.SemaphoreType.DMA((N_DEV - 1,)),
            pltpu.SemaphoreType.DMA((N_DEV - 1,)),
            pltpu.SemaphoreType.DMA((N_DEV - 1,)),
            pltpu.SemaphoreType.DMA((N_DEV - 1,)),
            pltpu.SemaphoreType.DMA((3,)),
        ],
        compiler_params=pltpu.CompilerParams(
            collective_id=0,
            vmem_limit_bytes=56 * 1024 * 1024,
        ),
    )(q, kh, vh, bias)

    wob = Wo.astype(jnp.bfloat16)
    out = lax.dot_general(ctx, wob, (((1,), (0,)), ((), ())),
                          preferred_element_type=jnp.float32)
    return out.reshape(1, S, HD)


# device time: 411420 ns/iter; 1.1090x vs baseline; 1.1090x over previous
import jax
import jax.numpy as jnp
from jax import lax
from jax.experimental import pallas as pl
from jax.experimental.pallas import tpu as pltpu

N_DEV = 4
S = 2048
H = 8
HR = H // 2
D = 128
HD = H * D
T = 512
NT = S // T
BLK = 64
BPS = S // BLK
SCALE = 0.08838834764831843
NEG = -1e9


def _body(q_ref, k_ref, v_ref, bias_ref, ctx_ref,
          commR_k, commR_v, commL_k, commL_v, biasR, biasL,
          m_scr, l_scr, acc_scr,
          kRs, kRr, vRs, vRr, kLs, kLr, vLs, vLr, local_sems):
    my = lax.axis_index("i")
    left = lax.rem(my + N_DEV - 1, N_DEV)
    right = lax.rem(my + 1, N_DEV)

    barrier = pltpu.get_barrier_semaphore()
    for nbr in (left, right):
        pl.semaphore_signal(barrier, inc=1, device_id=(nbr,),
                            device_id_type=pl.DeviceIdType.MESH)
    pl.semaphore_wait(barrier, 2)

    copies = [
        pltpu.make_async_copy(k_ref.at[0:HR], commR_k.at[0], local_sems.at[0]),
        pltpu.make_async_copy(v_ref.at[0:HR], commR_v.at[0], local_sems.at[1]),
        pltpu.make_async_copy(k_ref.at[HR:H], commL_k.at[0], local_sems.at[2]),
        pltpu.make_async_copy(v_ref.at[HR:H], commL_v.at[0], local_sems.at[3]),
        pltpu.make_async_copy(bias_ref.at[my], biasR, local_sems.at[4]),
        pltpu.make_async_copy(bias_ref.at[my], biasL, local_sems.at[5]),
    ]
    for c in copies:
        c.start()
    for c in copies:
        c.wait()

    m_scr[...] = jnp.full((S, H), -1e30, jnp.float32)
    l_scr[...] = jnp.zeros((S, H), jnp.float32)
    acc_scr[...] = jnp.zeros((S, HD), jnp.float32)

    for s in range(N_DEV):
        rdmas = []
        if s < N_DEV - 1:
            snd, rcv = s % 2, (s + 1) % 2
            for src, sem_s, sem_r, dev in (
                    (commR_k, kRs, kRr, right), (commR_v, vRs, vRr, right),
                    (commL_k, kLs, kLr, left), (commL_v, vLs, vLr, left)):
                r = pltpu.make_async_remote_copy(
                    src_ref=src.at[snd], dst_ref=src.at[rcv],
                    send_sem=sem_s.at[s], recv_sem=sem_r.at[s],
                    device_id=(dev,), device_id_type=pl.DeviceIdType.MESH)
                r.start()
                rdmas.append(r)

        slot = 0 if s == 0 else s % 2
        for h in range(H):
            if h < HR:
                comm_k, comm_v, bias_vm, hh = commR_k, commR_v, biasR, h
            else:
                comm_k, comm_v, bias_vm, hh = commL_k, commL_v, biasL, h - HR

            def blk(t, carry, slot=slot, h=h, hh=hh,
                    comm_k=comm_k, comm_v=comm_v, bias_vm=bias_vm):
                rows = pl.ds(t * T, T)
                q_t = q_ref[h, rows, :]
                k_h = comm_k[slot, hh]
                v_h = comm_v[slot, hh]
                sc = lax.dot_general(
                    q_t, k_h, (((1,), (1,)), ((), ())),
                    preferred_element_type=jnp.float32)
                sc = sc * SCALE + bias_vm[rows, :].astype(jnp.float32)
                mx = jnp.max(sc, axis=1, keepdims=True)
                m_old = m_scr[rows, h:h + 1]
                m_new = jnp.maximum(m_old, mx)
                alpha = jnp.exp(m_old - m_new)
                w = jnp.exp(sc - m_new)
                l_new = (l_scr[rows, h:h + 1] * alpha
                         + jnp.sum(w, axis=1, keepdims=True))
                a_new = acc_scr[rows, h * D:(h + 1) * D] * alpha + \
                    lax.dot_general(
                        w.astype(jnp.bfloat16), v_h,
                        (((1,), (0,)), ((), ())),
                        preferred_element_type=jnp.float32)
                m_scr[rows, h:h + 1] = m_new
                l_scr[rows, h:h + 1] = l_new
                acc_scr[rows, h * D:(h + 1) * D] = a_new
                return carry

            lax.fori_loop(0, NT, blk, 0)

        if s < N_DEV - 1:
            for r in rdmas:
                r.wait()
            pR = lax.rem(my - (s + 1) + N_DEV, N_DEV)
            pL = lax.rem(my + (s + 1), N_DEV)
            cbR = pltpu.make_async_copy(bias_ref.at[pR], biasR,
                                        local_sems.at[4])
            cbL = pltpu.make_async_copy(bias_ref.at[pL], biasL,
                                        local_sems.at[5])
            cbR.start()
            cbL.start()
            cbR.wait()
            cbL.wait()

    for h in range(H):
        inv = 1.0 / l_scr[:, h:h + 1]
        ctx_ref[:, h * D:(h + 1) * D] = (
            acc_scr[:, h * D:(h + 1) * D] * inv).astype(jnp.bfloat16)


def kernel(x, Wq, K_ext, V_ext, Wo):
    xb = x[0].astype(jnp.bfloat16)
    wqb = Wq.astype(jnp.bfloat16)
    q = lax.dot_general(xb, wqb, (((1,), (0,)), ((), ())),
                        preferred_element_type=jnp.float32)
    q = q.astype(jnp.bfloat16).reshape(S, H, D).transpose(1, 0, 2)
    kh = K_ext[0].astype(jnp.bfloat16).transpose(1, 0, 2)
    vh = V_ext[0].astype(jnp.bfloat16).transpose(1, 0, 2)

    my = lax.axis_index("i")
    qb = my * BPS + jnp.arange(S, dtype=jnp.int32) // BLK
    kb = (jnp.arange(N_DEV, dtype=jnp.int32)[:, None] * BPS
          + (jnp.arange(S, dtype=jnp.int32) // BLK)[None, :])
    qb2 = qb[None, :, None]
    kb2 = kb[:, None, :]
    mask = (qb2 == kb2) | (kb2 == 0) | ((qb2 + kb2) % 3 == 0)
    bias = jnp.where(mask, 0.0, NEG).astype(jnp.bfloat16)

    ctx = pl.pallas_call(
        _body,
        out_shape=jax.ShapeDtypeStruct((S, HD), jnp.bfloat16),
        in_specs=[
            pl.BlockSpec(memory_space=pltpu.VMEM),
            pl.BlockSpec(memory_space=pl.ANY),
            pl.BlockSpec(memory_space=pl.ANY),
            pl.BlockSpec(memory_space=pl.ANY),
        ],
        out_specs=pl.BlockSpec(memory_space=pltpu.VMEM),
        scratch_shapes=[
            pltpu.VMEM((2, HR, S, D), jnp.bfloat16),
            pltpu.VMEM((2, HR, S, D), jnp.bfloat16),
            pltpu.VMEM((2, HR, S, D), jnp.bfloat16),
            pltpu.VMEM((2, HR, S, D), jnp.bfloat16),
            pltpu.VMEM((S, S), jnp.bfloat16),
            pltpu.VMEM((S, S), jnp.bfloat16),
            pltpu.VMEM((S, H), jnp.float32),
            pltpu.VMEM((S, H), jnp.float32),
            pltpu.VMEM((S, HD), jnp.float32),
            pltpu.SemaphoreType.DMA((N_DEV - 1,)),
            pltpu.SemaphoreType.DMA((N_DEV - 1,)),
            pltpu.SemaphoreType.DMA((N_DEV - 1,)),
            pltpu.SemaphoreType.DMA((N_DEV - 1,)),
            pltpu.SemaphoreType.DMA((N_DEV - 1,)),
            pltpu.SemaphoreType.DMA((N_DEV - 1,)),
            pltpu.SemaphoreType.DMA((N_DEV - 1,)),
            pltpu.SemaphoreType.DMA((N_DEV - 1,)),
            pltpu.SemaphoreType.DMA((6,)),
        ],
        compiler_params=pltpu.CompilerParams(
            collective_id=0,
            vmem_limit_bytes=56 * 1024 * 1024,
        ),
    )(q, kh, vh, bias)

    wob = Wo.astype(jnp.bfloat16)
    out = lax.dot_general(ctx, wob, (((1,), (0,)), ((), ())),
                          preferred_element_type=jnp.float32)
    return out.reshape(1, S, HD)
